# baseline (speedup 1.0000x reference)
import functools
import math

import jax
import jax.numpy as jnp
from jax import lax
from jax.experimental import pallas as pl
from jax.experimental.pallas import tpu as pltpu
from jax.experimental.pallas import tpu_sc as plsc

B, T, D, MAX_LEN = 4, 8192, 768, 8192
EPS = 1e-5
SCALE = math.sqrt(float(D))
N = B * T

ROWS_BLK = 512


def _ln_body(pe_ref, w_ref, b_ref, out_ref):
    y = pe_ref[...] * SCALE
    mu = jnp.mean(y, axis=-1, keepdims=True)
    yc = y - mu
    var = jnp.mean(yc * yc, axis=-1, keepdims=True)
    out_ref[...] = yc * lax.rsqrt(var + EPS) * w_ref[...] + b_ref[...]


def _normalize_table(pe, ln_w, ln_b):
    return pl.pallas_call(
        _ln_body,
        grid=(MAX_LEN // ROWS_BLK,),
        in_specs=[
            pl.BlockSpec((ROWS_BLK, D), lambda i: (i, 0)),
            pl.BlockSpec((1, D), lambda i: (0, 0)),
            pl.BlockSpec((1, D), lambda i: (0, 0)),
        ],
        out_specs=pl.BlockSpec((ROWS_BLK, D), lambda i: (i, 0)),
        out_shape=jax.ShapeDtypeStruct((MAX_LEN, D), jnp.float32),
    )(pe, ln_w.reshape(1, D), ln_b.reshape(1, D))


_NC, _NS = 2, 16
NW = _NC * _NS
RPW = N // NW
CHUNK = 32
NCHUNK = RPW // CHUNK
LPR = D // 16


@functools.cache
def _make_gather_add():
    mesh = plsc.VectorSubcoreMesh(core_axis_name="c", subcore_axis_name="s",
                                  num_cores=_NC, num_subcores=_NS)

    @functools.partial(
        pl.kernel,
        out_type=jax.ShapeDtypeStruct((N, D), jnp.float32),
        mesh=mesh,
        scratch_types=[
            pltpu.VMEM((NCHUNK, CHUNK), jnp.int32),
            pltpu.VMEM((2, CHUNK, D), jnp.float32),
            pltpu.VMEM((2, CHUNK, D), jnp.float32),
            pltpu.SemaphoreType.DMA,
            pltpu.SemaphoreType.DMA,
            pltpu.SemaphoreType.DMA,
            pltpu.SemaphoreType.DMA,
            pltpu.SemaphoreType.DMA,
            pltpu.SemaphoreType.DMA,
        ],
    )
    def gather_add(table_hbm, idx_hbm, x_hbm, out_hbm,
                   idx_all, xb, gb, sg0, sg1, sx0, sx1, so0, so1):
        sg, sx, so = (sg0, sg1), (sx0, sx1), (so0, so1)
        wid = lax.axis_index("s") * _NC + lax.axis_index("c")
        base = pl.multiple_of(wid * RPW, CHUNK)

        def row_off(k):
            return pl.multiple_of(base + k * CHUNK, CHUNK)

        def start_gx(b, k):
            pltpu.async_copy(table_hbm.at[idx_all.at[k]], gb.at[b], sg[b])
            pltpu.async_copy(x_hbm.at[pl.ds(row_off(k), CHUNK)],
                             xb.at[b], sx[b])

        def wait_gx(b, k):
            pltpu.make_async_copy(table_hbm.at[idx_all.at[k]],
                                  gb.at[b], sg[b]).wait()
            pltpu.make_async_copy(x_hbm.at[pl.ds(row_off(k), CHUNK)],
                                  xb.at[b], sx[b]).wait()

        def start_out(b, k):
            pltpu.async_copy(xb.at[b], out_hbm.at[pl.ds(row_off(k), CHUNK)],
                             so[b])

        def wait_out(b, k):
            pltpu.make_async_copy(xb.at[b],
                                  out_hbm.at[pl.ds(row_off(k), CHUNK)],
                                  so[b]).wait()

        def accum(b):
            def row(i, c):
                for j in range(LPR):
                    sl = pl.ds(j * 16, 16)
                    xb[b, i, sl] = xb[b, i, sl] + gb[b, i, sl]
                return c

            lax.fori_loop(0, CHUNK, row, 0)

        pltpu.sync_copy(idx_hbm.at[wid], idx_all)
        start_gx(0, 0)

        @pl.loop(0, NCHUNK, step=2)
        def pair(k):
            @pl.when(k > 0)
            def _():
                wait_out(1, k - 1)

            start_gx(1, k + 1)
            wait_gx(0, k)
            accum(0)
            start_out(0, k)
            wait_gx(1, k + 1)
            accum(1)

            @pl.when(k + 2 < NCHUNK)
            def _():
                wait_out(0, k)
                start_gx(0, k + 2)

            start_out(1, k + 1)

        wait_out(0, NCHUNK - 2)
        wait_out(1, NCHUNK - 1)

    return gather_add


def kernel(x, positions, pe, ln_w, ln_b):
    pe_norm = _normalize_table(pe, ln_w, ln_b)
    idx = positions.reshape(NW, NCHUNK, CHUNK).astype(jnp.int32)
    out = _make_gather_add()(pe_norm, idx, x.reshape(N, D))
    return out.reshape(B, T, D)


# speedup vs baseline: 1.1223x; 1.1223x over previous
import functools
import math

import jax
import jax.numpy as jnp
from jax import lax
from jax.experimental import pallas as pl
from jax.experimental.pallas import tpu as pltpu
from jax.experimental.pallas import tpu_sc as plsc

B, T, D, MAX_LEN = 4, 8192, 768, 8192
EPS = 1e-5
SCALE = math.sqrt(float(D))
N = B * T

ROWS_BLK = 512


def _ln_body(pe_ref, w_ref, b_ref, out_ref):
    y = pe_ref[...] * SCALE
    mu = jnp.mean(y, axis=-1, keepdims=True)
    yc = y - mu
    var = jnp.mean(yc * yc, axis=-1, keepdims=True)
    out_ref[...] = yc * lax.rsqrt(var + EPS) * w_ref[...] + b_ref[...]


def _normalize_table(pe, ln_w, ln_b):
    return pl.pallas_call(
        _ln_body,
        grid=(MAX_LEN // ROWS_BLK,),
        in_specs=[
            pl.BlockSpec((ROWS_BLK, D), lambda i: (i, 0)),
            pl.BlockSpec((1, D), lambda i: (0, 0)),
            pl.BlockSpec((1, D), lambda i: (0, 0)),
        ],
        out_specs=pl.BlockSpec((ROWS_BLK, D), lambda i: (i, 0)),
        out_shape=jax.ShapeDtypeStruct((MAX_LEN, D), jnp.float32),
    )(pe, ln_w.reshape(1, D), ln_b.reshape(1, D))


_NC, _NS = 2, 16
NW = _NC * _NS
RPW = N // NW
CHUNK = 32
NCHUNK = RPW // CHUNK
LPR = D // 16


@functools.cache
def _make_gather_add():
    mesh = plsc.VectorSubcoreMesh(core_axis_name="c", subcore_axis_name="s",
                                  num_cores=_NC, num_subcores=_NS)

    @functools.partial(
        pl.kernel,
        out_type=jax.ShapeDtypeStruct((N, D), jnp.float32),
        mesh=mesh,
        scratch_types=[
            pltpu.VMEM((NCHUNK, CHUNK), jnp.int32),
            pltpu.VMEM((2, CHUNK, D), jnp.float32),
            pltpu.VMEM((2, CHUNK, D), jnp.float32),
            pltpu.SemaphoreType.DMA,
            pltpu.SemaphoreType.DMA,
            pltpu.SemaphoreType.DMA,
            pltpu.SemaphoreType.DMA,
            pltpu.SemaphoreType.DMA,
            pltpu.SemaphoreType.DMA,
        ],
    )
    def gather_add(table_hbm, idx_hbm, x_hbm, out_hbm,
                   idx_all, xb, gb, sg0, sg1, sx0, sx1, so0, so1):
        sg, sx, so = (sg0, sg1), (sx0, sx1), (so0, so1)
        wid = lax.axis_index("s") * _NC + lax.axis_index("c")
        base = pl.multiple_of(wid * RPW, CHUNK)

        def row_off(k):
            return pl.multiple_of(base + k * CHUNK, CHUNK)

        def start_gx(b, k):
            pltpu.async_copy(table_hbm.at[idx_all.at[k]], gb.at[b], sg[b])
            pltpu.async_copy(x_hbm.at[pl.ds(row_off(k), CHUNK)],
                             xb.at[b], sx[b])

        def wait_gx(b, k):
            pltpu.make_async_copy(table_hbm.at[idx_all.at[k]],
                                  gb.at[b], sg[b]).wait()
            pltpu.make_async_copy(x_hbm.at[pl.ds(row_off(k), CHUNK)],
                                  xb.at[b], sx[b]).wait()

        def start_out(b, k):
            pltpu.async_copy(xb.at[b], out_hbm.at[pl.ds(row_off(k), CHUNK)],
                             so[b])

        def wait_out(b, k):
            pltpu.make_async_copy(xb.at[b],
                                  out_hbm.at[pl.ds(row_off(k), CHUNK)],
                                  so[b]).wait()

        def accum(b):
            def row(i, c):
                for j in range(LPR):
                    sl = pl.ds(j * 16, 16)
                    plsc.addupdate(xb.at[b, i, sl], gb[b, i, sl])
                return c

            lax.fori_loop(0, CHUNK, row, 0)

        pltpu.sync_copy(idx_hbm.at[wid], idx_all)
        start_gx(0, 0)

        @pl.loop(0, NCHUNK, step=2)
        def pair(k):
            @pl.when(k > 0)
            def _():
                wait_out(1, k - 1)

            start_gx(1, k + 1)
            wait_gx(0, k)
            accum(0)
            start_out(0, k)
            wait_gx(1, k + 1)
            accum(1)

            @pl.when(k + 2 < NCHUNK)
            def _():
                wait_out(0, k)
                start_gx(0, k + 2)

            start_out(1, k + 1)

        wait_out(0, NCHUNK - 2)
        wait_out(1, NCHUNK - 1)

    return gather_add


def kernel(x, positions, pe, ln_w, ln_b):
    pe_norm = _normalize_table(pe, ln_w, ln_b)
    idx = positions.reshape(NW, NCHUNK, CHUNK).astype(jnp.int32)
    out = _make_gather_add()(pe_norm, idx, x.reshape(N, D))
    return out.reshape(B, T, D)
